# baseline (speedup 1.0000x reference)
import dataclasses
import functools

import jax
import jax.numpy as jnp
from jax import lax
from jax.experimental import pallas as pl
from jax.experimental.pallas import tpu as pltpu
from jax.experimental.pallas import tpu_sc as plsc

D = 128
N0, N1 = 10000, 2500
E0, E1 = 320000, 80000
L = 2

NTILES = 32
CHUNK = 128

NP0 = 10112
NP1 = 2560
EP0 = 327680
EP1 = 81920
CPT0 = EP0 // (NTILES * CHUNK)
CPT1 = EP1 // (NTILES * CHUNK)
CA0, CB0 = 120, 40
CA1, CB1 = 30, 10
NCH0 = NP0 // CHUNK
REPS0, REPS1 = 4, 8


def _vector_mesh():
    return plsc.VectorSubcoreMesh(core_axis_name="c", subcore_axis_name="s")


def _i32(x):
    return jnp.int32(x)


def _sc_compiler_params():
    cp = pltpu.CompilerParams()
    if "needs_layout_passes" in pltpu.CompilerParams.__dataclass_fields__:
        cp = dataclasses.replace(cp, needs_layout_passes=False)
    return cp


def _loop(n, body):
    def wrapped(i, carry):
        body(i)
        return carry

    lax.fori_loop(_i32(0), _i32(n), wrapped, None)


def _ln(h):
    mu = jnp.mean(h, axis=-1, keepdims=True)
    var = jnp.mean((h - mu) ** 2, axis=-1, keepdims=True)
    return (h - mu) * lax.rsqrt(var + 1e-5)


def _ln_body(h_ref, o_ref):
    o_ref[...] = _ln(h_ref[...])


def tc_ln(h):
    return pl.pallas_call(
        _ln_body,
        out_shape=jax.ShapeDtypeStruct(h.shape, jnp.float32),
    )(h)


def _ms_body(reps, h_ref, wm_ref, ws_ref, m_ref, s_ref):
    h = h_ref[...]
    n = h.shape[0]
    m = jnp.dot(h, wm_ref[...], preferred_element_type=jnp.float32, precision=lax.Precision.HIGHEST)
    for r in range(reps):
        m_ref[pl.ds(r * n, n), :] = m
    s_ref[...] = jnp.dot(h, ws_ref[...], preferred_element_type=jnp.float32, precision=lax.Precision.HIGHEST)


def tc_ms(h, wm, ws, reps):
    n = h.shape[0]
    return pl.pallas_call(
        functools.partial(_ms_body, reps),
        out_shape=[jax.ShapeDtypeStruct((reps * n, D), jnp.float32),
                   jax.ShapeDtypeStruct((n, D), jnp.float32)],
    )(h, wm, ws)


def _comb1_body(s_ref, a_ref, wt_ref, u_ref, hn_ref):
    h = s_ref[...] + a_ref[0] + a_ref[1]
    u_ref[...] = jnp.dot(h, wt_ref[...], preferred_element_type=jnp.float32, precision=lax.Precision.HIGHEST)
    hn_ref[...] = _ln(h)


def tc_comb1(s, agg, wtop):
    n = s.shape[0]
    return pl.pallas_call(
        _comb1_body,
        out_shape=[jax.ShapeDtypeStruct((n, D), jnp.float32),
                   jax.ShapeDtypeStruct((n, D), jnp.float32)],
    )(s, agg, wtop)


UPF_R = 632


def _upfin_body(s_ref, a_ref, wb_ref, bias_ref, u_ref, w_ref, o_ref):
    h = s_ref[...] + a_ref[0] + a_ref[1]
    h0c = (h + jnp.dot(h, wb_ref[...], preferred_element_type=jnp.float32,
                       precision=lax.Precision.HIGHEST) + bias_ref[...])
    wcol = w_ref[...]
    onehot = (wcol == lax.broadcasted_iota(jnp.int32, (UPF_R, NP1), 1)
              ).astype(jnp.float32)
    inp = jnp.dot(onehot, u_ref[...], preferred_element_type=jnp.float32)
    o_ref[...] = _ln(h0c + inp)


def tc_upfin(s0, agg0, wbot, bias_row, u1pad, win_col):
    grid = NP0 // UPF_R
    return pl.pallas_call(
        _upfin_body,
        grid=(grid,),
        in_specs=[
            pl.BlockSpec((UPF_R, D), lambda i: (i, jnp.int32(0))),
            pl.BlockSpec((2, UPF_R, D),
                         lambda i: (jnp.int32(0), i, jnp.int32(0))),
            pl.BlockSpec((D, D), lambda i: (jnp.int32(0), jnp.int32(0))),
            pl.BlockSpec((1, D), lambda i: (jnp.int32(0), jnp.int32(0))),
            pl.BlockSpec((NP1, D), lambda i: (jnp.int32(0), jnp.int32(0))),
            pl.BlockSpec((UPF_R, 1), lambda i: (i, jnp.int32(0))),
        ],
        out_specs=pl.BlockSpec((UPF_R, D), lambda i: (i, jnp.int32(0))),
        out_shape=jax.ShapeDtypeStruct((NP0, D), jnp.float32),
    )(s0, agg0, wbot, bias_row, u1pad, win_col)


NIP = 2512


def _seg_level(msg_hbm, s_hbm, d_hbm, ebase, acc_sh,
               sA, dA, sB, dB, bufA, bufB, semA, semB, ssems, npairs):
    ssemA, ssemB = ssems

    def idxcpy(k, s_v, d_v):
        b = ebase + k * _i32(CHUNK)
        pltpu.sync_copy(s_hbm.at[pl.ds(b, CHUNK)], s_v)
        pltpu.sync_copy(d_hbm.at[pl.ds(b, CHUNK)], d_v)

    def drain(buf, sem):
        pltpu.make_async_copy(msg_hbm.at[pl.ds(0, CHUNK)], buf, sem).wait()

    idxcpy(_i32(0), sA, dA)
    pltpu.async_copy(msg_hbm.at[sA], bufA, semA)

    def pair(p):
        k0 = p * _i32(2)
        drain(bufA, semA)
        pltpu.async_copy(bufA, acc_sh.at[dA], ssemA, add=True)
        @pl.when(p > _i32(0))
        def _():
            drain(bufB, ssemB)
        idxcpy(k0 + _i32(1), sB, dB)
        pltpu.async_copy(msg_hbm.at[sB], bufB, semB)
        @pl.when(p < npairs - _i32(1))
        def _():
            drain(bufA, ssemA)
            idxcpy(k0 + _i32(2), sA, dA)
            pltpu.async_copy(msg_hbm.at[sA], bufA, semA)

        drain(bufB, semB)
        pltpu.async_copy(bufB, acc_sh.at[dB], ssemB, add=True)

    def wrapped(i, carry):
        pair(i)
        return carry

    lax.fori_loop(_i32(0), npairs, wrapped, None)
    drain(bufA, ssemA)
    drain(bufB, ssemB)


def _winmap_body(idx_hbm, win_hbm, idx_v, win_v):
    pltpu.sync_copy(idx_hbm, idx_v)

    def init(t):
        win_v[pl.ds(t * _i32(16), 16)] = jnp.full((16,), N1, jnp.int32)

    _loop(NP0 // 16, init)

    lanes = lax.iota(jnp.int32, 16)

    def group(t):
        j0 = t * _i32(16)
        idxvec = idx_v[pl.ds(j0, 16)]
        jvec = lanes + j0
        for lane in range(16):
            plsc.store_scatter(win_v, [idxvec], jvec, mask=lanes == _i32(lane))

    _loop(NIP // 16, group)

    pltpu.sync_copy(win_v, win_hbm)


def _zero_buf(buf):
    zz = jnp.zeros((16,), jnp.float32)

    def z(i):
        r = i // _i32(8)
        c = (i % _i32(8)) * _i32(16)
        buf[r, pl.ds(c, 16)] = zz

    _loop(CHUNK * 8, z)


def _zero_stripe(acc, off, stripe, zrows):
    full, rem = stripe // CHUNK, stripe % CHUNK
    for q in range(full):
        pltpu.sync_copy(zrows, acc.at[pl.ds(off + _i32(q * CHUNK), CHUNK)])
    if rem:
        pltpu.sync_copy(zrows.at[pl.ds(0, rem)],
                        acc.at[pl.ds(off + _i32(full * CHUNK), rem)])


def sc_layer_seg(m0, m1, src0, dst0, src1, dst1, idx1_pad, with_win):
    stripe0, stripe1 = NP0 // 16, NP1 // 16

    out_type = [jax.ShapeDtypeStruct((2, NP0, D), jnp.float32),
                jax.ShapeDtypeStruct((2, NP1, D), jnp.float32)]
    scratch = [
        pltpu.VMEM((CHUNK,), jnp.int32),
        pltpu.VMEM((CHUNK,), jnp.int32),
        pltpu.VMEM((CHUNK,), jnp.int32),
        pltpu.VMEM((CHUNK,), jnp.int32),
        pltpu.VMEM((CHUNK, D), jnp.float32),
        pltpu.VMEM((CHUNK, D), jnp.float32),
        pltpu.VMEM_SHARED((NP0, D), jnp.float32),
        pltpu.SemaphoreType.DMA,
        pltpu.SemaphoreType.DMA,
        pltpu.SemaphoreType.DMA,
        pltpu.SemaphoreType.DMA,
    ]
    if with_win:
        out_type.append(jax.ShapeDtypeStruct((NP0,), jnp.int32))
        scratch += [pltpu.VMEM((NIP,), jnp.int32),
                    pltpu.VMEM((NP0,), jnp.int32)]

    @functools.partial(
        pl.kernel,
        out_type=out_type,
        mesh=_vector_mesh(),
        scratch_types=scratch,
        compiler_params=_sc_compiler_params(),
    )
    def k(m0_hbm, m1_hbm, s0_hbm, d0_hbm, s1_hbm, d1_hbm,
          idx_hbm, a0_hbm, a1_hbm, *rest):
        if with_win:
            win_hbm, sA, dA, sB, dB, bufA, bufB, acc, semA, semB, \
                ssemA, ssemB, idx_v, win_v = rest
        else:
            sA, dA, sB, dB, bufA, bufB, acc, semA, semB, ssemA, ssemB = rest
        cid = lax.axis_index("c")
        sid = lax.axis_index("s")
        off0 = sid * _i32(stripe0)
        off1 = sid * _i32(stripe1)
        _zero_buf(bufA)
        _zero_stripe(acc, off0, stripe0, bufA)
        plsc.subcore_barrier()

        if with_win:
            @pl.when(jnp.logical_and(cid == _i32(0), sid == _i32(0)))
            def _():
                _winmap_body(idx_hbm, win_hbm, idx_v, win_v)

        is0 = cid == _i32(0)
        ebase0 = jnp.where(is0, sid * _i32(CA0),
                           _i32(16 * CA0) + sid * _i32(CB0)) * _i32(CHUNK)
        np0 = jnp.where(is0, _i32(CA0 // 2), _i32(CB0 // 2))
        _seg_level(m0_hbm, s0_hbm, d0_hbm, ebase0, acc,
                   sA, dA, sB, dB, bufA, bufB, semA, semB, (ssemA, ssemB),
                   np0)

        plsc.subcore_barrier()
        pltpu.sync_copy(acc.at[pl.ds(off0, stripe0)],
                        a0_hbm.at[cid, pl.ds(off0, stripe0)])
        plsc.subcore_barrier()
        _zero_buf(bufA)
        _zero_stripe(acc, off1, stripe1, bufA)
        plsc.subcore_barrier()

        ebase1 = jnp.where(is0, sid * _i32(CA1),
                           _i32(16 * CA1) + sid * _i32(CB1)) * _i32(CHUNK)
        np1 = jnp.where(is0, _i32(CA1 // 2), _i32(CB1 // 2))
        _seg_level(m1_hbm, s1_hbm, d1_hbm, ebase1, acc,
                   sA, dA, sB, dB, bufA, bufB, semA, semB, (ssemA, ssemB),
                   np1)

        plsc.subcore_barrier()
        pltpu.sync_copy(acc.at[pl.ds(off1, stripe1)],
                        a1_hbm.at[cid, pl.ds(off1, stripe1)])

    return k(m0, m1, src0, dst0, src1, dst1, idx1_pad)


def sc_upgather(u1pad, win):
    @functools.partial(
        pl.kernel,
        out_type=jax.ShapeDtypeStruct((NP0, D), jnp.float32),
        mesh=_vector_mesh(),
        scratch_types=[
            pltpu.VMEM((CHUNK,), jnp.int32),
            pltpu.VMEM((CHUNK, D), jnp.float32),
            pltpu.SemaphoreType.DMA,
        ],
    )
    def k(u_hbm, win_hbm, out_hbm, widx_v, rows_v, sem):
        cid = lax.axis_index("c")
        sid = lax.axis_index("s")
        wid = cid * _i32(16) + sid
        for kk in range(3):
            ch = wid + _i32(kk * NTILES)

            @pl.when(ch < _i32(NCH0))
            def _():
                base = ch * _i32(CHUNK)
                pltpu.sync_copy(win_hbm.at[pl.ds(base, CHUNK)], widx_v)
                pltpu.async_copy(u_hbm.at[widx_v], rows_v, sem).wait()
                pltpu.sync_copy(rows_v, out_hbm.at[pl.ds(base, CHUNK)])

    return k(u1pad, win)


def kernel(hn0, hn1, Wself, Wmsg, Wup, bup, edge_index0, edge_index1, idx1):
    f32 = jnp.float32
    i32 = jnp.int32

    h0 = jnp.pad(hn0.astype(f32), ((0, NP0 - N0), (0, 0)))
    h1 = jnp.pad(hn1.astype(f32), ((0, NP1 - N1), (0, 0)))

    src0 = jnp.pad(edge_index0[0].astype(i32),
                   (0, EP0 - E0))
    dst0 = jnp.pad(edge_index0[1].astype(i32), (0, EP0 - E0),
                   constant_values=NP0 - 1)
    src1 = jnp.pad(edge_index1[0].astype(i32),
                   (0, EP1 - E1))
    dst1 = jnp.pad(edge_index1[1].astype(i32), (0, EP1 - E1),
                   constant_values=NP1 - 1)
    src0 = src0 + (jnp.arange(EP0, dtype=i32) // CHUNK % REPS0) * NP0
    src1 = src1 + (jnp.arange(EP1, dtype=i32) // CHUNK % REPS1) * NP1
    idx1_pad = jnp.pad(idx1.astype(i32), (0, NIP - N1), constant_values=N0)

    Wself = Wself.astype(f32)
    Wmsg = Wmsg.astype(f32)
    Wup = Wup.astype(f32)
    bup = bup.astype(f32)

    h0 = tc_ln(h0)
    h1 = tc_ln(h1)

    win = None
    for l in range(L):
        m0r, s0 = tc_ms(h0, Wmsg[l, 0], Wself[l, 0], REPS0)
        m1r, s1 = tc_ms(h1, Wmsg[l, 1], Wself[l, 1], REPS1)
        if l == 0:
            agg0, agg1, win = sc_layer_seg(m0r, m1r, src0, dst0, src1, dst1,
                                           idx1_pad, with_win=True)
        else:
            agg0, agg1 = sc_layer_seg(m0r, m1r, src0, dst0, src1, dst1,
                                      idx1_pad, with_win=False)

        if l == 0:
            win_col = jnp.reshape(win, (NP0, 1))
        u1, h1 = tc_comb1(s1, agg1, Wup[l, :D])
        h0 = tc_upfin(s0, agg0, Wup[l, D:], jnp.reshape(bup[l], (1, D)),
                      u1, win_col)

    return (h0[:N0].astype(jnp.float64), h1[:N1].astype(jnp.float64))


# speedup vs baseline: 1.0791x; 1.0791x over previous
import dataclasses
import functools

import jax
import jax.numpy as jnp
from jax import lax
from jax.experimental import pallas as pl
from jax.experimental.pallas import tpu as pltpu
from jax.experimental.pallas import tpu_sc as plsc

D = 128
N0, N1 = 10000, 2500
E0, E1 = 320000, 80000
L = 2

NTILES = 32
CHUNK = 128

NP0 = 10112
NP1 = 2560
EP0 = 327680
EP1 = 81920
CPT0 = EP0 // (NTILES * CHUNK)
CPT1 = EP1 // (NTILES * CHUNK)
CA0, CB0 = 120, 40
CA1, CB1 = 30, 10
NCH0 = NP0 // CHUNK
REPS0, REPS1 = 4, 8


def _vector_mesh():
    return plsc.VectorSubcoreMesh(core_axis_name="c", subcore_axis_name="s")


def _i32(x):
    return jnp.int32(x)


def _sc_compiler_params():
    cp = pltpu.CompilerParams()
    if "needs_layout_passes" in pltpu.CompilerParams.__dataclass_fields__:
        cp = dataclasses.replace(cp, needs_layout_passes=False)
    return cp


def _loop(n, body):
    def wrapped(i, carry):
        body(i)
        return carry

    lax.fori_loop(_i32(0), _i32(n), wrapped, None)


def _ln(h):
    mu = jnp.mean(h, axis=-1, keepdims=True)
    var = jnp.mean((h - mu) ** 2, axis=-1, keepdims=True)
    return (h - mu) * lax.rsqrt(var + 1e-5)


def _ln_body(h_ref, o_ref):
    o_ref[...] = _ln(h_ref[...])


def tc_ln(h):
    return pl.pallas_call(
        _ln_body,
        out_shape=jax.ShapeDtypeStruct(h.shape, jnp.float32),
    )(h)


def _ms_body(reps, h_ref, wm_ref, ws_ref, m_ref, s_ref):
    h = h_ref[...]
    n = h.shape[0]
    m = jnp.dot(h, wm_ref[...], preferred_element_type=jnp.float32, precision=lax.Precision.HIGHEST)
    for r in range(reps):
        m_ref[pl.ds(r * n, n), :] = m
    s_ref[...] = jnp.dot(h, ws_ref[...], preferred_element_type=jnp.float32, precision=lax.Precision.HIGHEST)


def tc_ms(h, wm, ws, reps):
    n = h.shape[0]
    return pl.pallas_call(
        functools.partial(_ms_body, reps),
        out_shape=[jax.ShapeDtypeStruct((reps * n, D), jnp.float32),
                   jax.ShapeDtypeStruct((n, D), jnp.float32)],
    )(h, wm, ws)


def _comb1_body(s_ref, a_ref, wt_ref, u_ref, hn_ref):
    h = s_ref[...] + a_ref[0] + a_ref[1]
    u_ref[...] = jnp.dot(h, wt_ref[...], preferred_element_type=jnp.float32, precision=lax.Precision.HIGHEST)
    hn_ref[...] = _ln(h)


def tc_comb1(s, agg, wtop):
    n = s.shape[0]
    return pl.pallas_call(
        _comb1_body,
        out_shape=[jax.ShapeDtypeStruct((n, D), jnp.float32),
                   jax.ShapeDtypeStruct((n, D), jnp.float32)],
    )(s, agg, wtop)


UPF_R = 632


def _upfin_body(s_ref, a_ref, wb_ref, bias_ref, u_ref, w_ref, o_ref):
    h = s_ref[...] + a_ref[0] + a_ref[1]
    h0c = (h + jnp.dot(h, wb_ref[...], preferred_element_type=jnp.float32,
                       precision=lax.Precision.HIGHEST) + bias_ref[...])
    wcol = w_ref[...]
    onehot = (wcol == lax.broadcasted_iota(jnp.int32, (UPF_R, NP1), 1)
              ).astype(jnp.float32)
    inp = jnp.dot(onehot, u_ref[...], preferred_element_type=jnp.float32)
    o_ref[...] = _ln(h0c + inp)


def tc_upfin(s0, agg0, wbot, bias_row, u1pad, win_col):
    grid = NP0 // UPF_R
    return pl.pallas_call(
        _upfin_body,
        grid=(grid,),
        in_specs=[
            pl.BlockSpec((UPF_R, D), lambda i: (i, jnp.int32(0))),
            pl.BlockSpec((2, UPF_R, D),
                         lambda i: (jnp.int32(0), i, jnp.int32(0))),
            pl.BlockSpec((D, D), lambda i: (jnp.int32(0), jnp.int32(0))),
            pl.BlockSpec((1, D), lambda i: (jnp.int32(0), jnp.int32(0))),
            pl.BlockSpec((NP1, D), lambda i: (jnp.int32(0), jnp.int32(0))),
            pl.BlockSpec((UPF_R, 1), lambda i: (i, jnp.int32(0))),
        ],
        out_specs=pl.BlockSpec((UPF_R, D), lambda i: (i, jnp.int32(0))),
        out_shape=jax.ShapeDtypeStruct((NP0, D), jnp.float32),
    )(s0, agg0, wbot, bias_row, u1pad, win_col)


NIP = 2512


def _seg_level(msg_hbm, s_hbm, d_hbm, ebase, acc_sh,
               sA, dA, sB, dB, bufA, bufB, semA, semB, npairs):
    def idxcpy(k, s_v, d_v):
        b = ebase + k * _i32(CHUNK)
        pltpu.sync_copy(s_hbm.at[pl.ds(b, CHUNK)], s_v)
        pltpu.sync_copy(d_hbm.at[pl.ds(b, CHUNK)], d_v)

    def drain(buf, sem):
        pltpu.make_async_copy(msg_hbm.at[pl.ds(0, CHUNK)], buf, sem).wait()

    idxcpy(_i32(0), sA, dA)
    pltpu.async_copy(msg_hbm.at[sA], bufA, semA)

    def pair(p):
        k0 = p * _i32(2)
        idxcpy(k0 + _i32(1), sB, dB)
        pltpu.async_copy(msg_hbm.at[sB], bufB, semB)
        drain(bufA, semA)
        pltpu.sync_copy(bufA, acc_sh.at[dA], add=True)

        @pl.when(p < npairs - _i32(1))
        def _():
            idxcpy(k0 + _i32(2), sA, dA)
            pltpu.async_copy(msg_hbm.at[sA], bufA, semA)

        drain(bufB, semB)
        pltpu.sync_copy(bufB, acc_sh.at[dB], add=True)

    def wrapped(i, carry):
        pair(i)
        return carry

    lax.fori_loop(_i32(0), npairs, wrapped, None)


def _winmap_body(idx_hbm, win_hbm, idx_v, win_v):
    pltpu.sync_copy(idx_hbm, idx_v)

    def init(t):
        win_v[pl.ds(t * _i32(16), 16)] = jnp.full((16,), N1, jnp.int32)

    _loop(NP0 // 16, init)

    lanes = lax.iota(jnp.int32, 16)

    def group(t):
        j0 = t * _i32(16)
        idxvec = idx_v[pl.ds(j0, 16)]
        jvec = lanes + j0
        for lane in range(16):
            plsc.store_scatter(win_v, [idxvec], jvec, mask=lanes == _i32(lane))

    _loop(NIP // 16, group)

    pltpu.sync_copy(win_v, win_hbm)


def _zero_buf(buf):
    zz = jnp.zeros((16,), jnp.float32)

    def z(i):
        r = i // _i32(8)
        c = (i % _i32(8)) * _i32(16)
        buf[r, pl.ds(c, 16)] = zz

    _loop(CHUNK * 8, z)


def _zero_stripe(acc, off, stripe, zrows):
    full, rem = stripe // CHUNK, stripe % CHUNK
    for q in range(full):
        pltpu.sync_copy(zrows, acc.at[pl.ds(off + _i32(q * CHUNK), CHUNK)])
    if rem:
        pltpu.sync_copy(zrows.at[pl.ds(0, rem)],
                        acc.at[pl.ds(off + _i32(full * CHUNK), rem)])


def sc_layer_seg(m0, m1, src0, dst0, src1, dst1, idx1_pad, with_win):
    stripe0, stripe1 = NP0 // 16, NP1 // 16

    out_type = [jax.ShapeDtypeStruct((2, NP0, D), jnp.float32),
                jax.ShapeDtypeStruct((2, NP1, D), jnp.float32)]
    scratch = [
        pltpu.VMEM((CHUNK,), jnp.int32),
        pltpu.VMEM((CHUNK,), jnp.int32),
        pltpu.VMEM((CHUNK,), jnp.int32),
        pltpu.VMEM((CHUNK,), jnp.int32),
        pltpu.VMEM((CHUNK, D), jnp.float32),
        pltpu.VMEM((CHUNK, D), jnp.float32),
        pltpu.VMEM_SHARED((NP0, D), jnp.float32),
        pltpu.SemaphoreType.DMA,
        pltpu.SemaphoreType.DMA,
    ]
    if with_win:
        out_type.append(jax.ShapeDtypeStruct((NP0,), jnp.int32))
        scratch += [pltpu.VMEM((NIP,), jnp.int32),
                    pltpu.VMEM((NP0,), jnp.int32)]

    @functools.partial(
        pl.kernel,
        out_type=out_type,
        mesh=_vector_mesh(),
        scratch_types=scratch,
        compiler_params=_sc_compiler_params(),
    )
    def k(m0_hbm, m1_hbm, s0_hbm, d0_hbm, s1_hbm, d1_hbm,
          idx_hbm, a0_hbm, a1_hbm, *rest):
        if with_win:
            win_hbm, sA, dA, sB, dB, bufA, bufB, acc, semA, semB, \
                idx_v, win_v = rest
        else:
            sA, dA, sB, dB, bufA, bufB, acc, semA, semB = rest
        cid = lax.axis_index("c")
        sid = lax.axis_index("s")
        off0 = sid * _i32(stripe0)
        off1 = sid * _i32(stripe1)
        _zero_buf(bufA)
        _zero_stripe(acc, off0, stripe0, bufA)
        plsc.subcore_barrier()

        if with_win:
            @pl.when(jnp.logical_and(cid == _i32(0), sid == _i32(0)))
            def _():
                _winmap_body(idx_hbm, win_hbm, idx_v, win_v)

        is0 = cid == _i32(0)
        ebase0 = jnp.where(is0, sid * _i32(CA0),
                           _i32(16 * CA0) + sid * _i32(CB0)) * _i32(CHUNK)
        np0 = jnp.where(is0, _i32(CA0 // 2), _i32(CB0 // 2))
        _seg_level(m0_hbm, s0_hbm, d0_hbm, ebase0, acc,
                   sA, dA, sB, dB, bufA, bufB, semA, semB, np0)

        plsc.subcore_barrier()
        pltpu.sync_copy(acc.at[pl.ds(off0, stripe0)],
                        a0_hbm.at[cid, pl.ds(off0, stripe0)])
        plsc.subcore_barrier()
        _zero_buf(bufA)
        _zero_stripe(acc, off1, stripe1, bufA)
        plsc.subcore_barrier()

        ebase1 = jnp.where(is0, sid * _i32(CA1),
                           _i32(16 * CA1) + sid * _i32(CB1)) * _i32(CHUNK)
        np1 = jnp.where(is0, _i32(CA1 // 2), _i32(CB1 // 2))
        _seg_level(m1_hbm, s1_hbm, d1_hbm, ebase1, acc,
                   sA, dA, sB, dB, bufA, bufB, semA, semB, np1)

        plsc.subcore_barrier()
        pltpu.sync_copy(acc.at[pl.ds(off1, stripe1)],
                        a1_hbm.at[cid, pl.ds(off1, stripe1)])

    return k(m0, m1, src0, dst0, src1, dst1, idx1_pad)


def kernel(hn0, hn1, Wself, Wmsg, Wup, bup, edge_index0, edge_index1, idx1):
    f32 = jnp.float32
    i32 = jnp.int32

    h0 = jnp.pad(hn0.astype(f32), ((0, NP0 - N0), (0, 0)))
    h1 = jnp.pad(hn1.astype(f32), ((0, NP1 - N1), (0, 0)))

    src0 = jnp.pad(edge_index0[0].astype(i32),
                   (0, EP0 - E0))
    dst0 = jnp.pad(edge_index0[1].astype(i32), (0, EP0 - E0),
                   constant_values=NP0 - 1)
    src1 = jnp.pad(edge_index1[0].astype(i32),
                   (0, EP1 - E1))
    dst1 = jnp.pad(edge_index1[1].astype(i32), (0, EP1 - E1),
                   constant_values=NP1 - 1)
    src0 = src0 + (jnp.arange(EP0, dtype=i32) // CHUNK % REPS0) * NP0
    src1 = src1 + (jnp.arange(EP1, dtype=i32) // CHUNK % REPS1) * NP1
    idx1_pad = jnp.pad(idx1.astype(i32), (0, NIP - N1), constant_values=N0)

    Wself = Wself.astype(f32)
    Wmsg = Wmsg.astype(f32)
    Wup = Wup.astype(f32)
    bup = bup.astype(f32)

    h0 = tc_ln(h0)
    h1 = tc_ln(h1)

    win = None
    for l in range(L):
        m0r, s0 = tc_ms(h0, Wmsg[l, 0], Wself[l, 0], REPS0)
        m1r, s1 = tc_ms(h1, Wmsg[l, 1], Wself[l, 1], REPS1)
        if l == 0:
            agg0, agg1, win = sc_layer_seg(m0r, m1r, src0, dst0, src1, dst1,
                                           idx1_pad, with_win=True)
        else:
            agg0, agg1 = sc_layer_seg(m0r, m1r, src0, dst0, src1, dst1,
                                      idx1_pad, with_win=False)

        if l == 0:
            win_col = jnp.reshape(win, (NP0, 1))
        u1, h1 = tc_comb1(s1, agg1, Wup[l, :D])
        h0 = tc_upfin(s0, agg0, Wup[l, D:], jnp.reshape(bup[l], (1, D)),
                      u1, win_col)

    return (h0[:N0].astype(jnp.float64), h1[:N1].astype(jnp.float64))
